# baseline (speedup 1.0000x reference)
import functools

import numpy as np

import jax
import jax.numpy as jnp
from jax import lax
from jax.experimental import pallas as pl
from jax.experimental.pallas import tpu as pltpu
from jax.experimental.pallas import tpu_sc as plsc

_N = 10000
_D = 128
_E = 320000
_NC = 2
_NS = 16
_C = 128
_CHUNKS = 80
_NBUF = 2
_NPK = 4
_GSPLIT = 4
_EPAD = _NC * _NS * _CHUNKS * _C
_NPAD = 10240
_RPT = _NPAD // _NS

_PERM = np.arange(_D).reshape(_D // 32, 2, 16).transpose(0, 2, 1).reshape(_D)

_mesh = plsc.VectorSubcoreMesh(core_axis_name="c", subcore_axis_name="s")


@functools.partial(
    pl.kernel,
    out_type=jax.ShapeDtypeStruct((_NC, _NPAD, _D), jnp.float32),
    mesh=_mesh,
    compiler_params=pltpu.CompilerParams(
        use_tc_tiling_on_sc=False, needs_layout_passes=False),
    scratch_types=[
        [pltpu.VMEM((_C, _D), jnp.bfloat16) for _ in range(_NBUF)],
        pltpu.VMEM((_C, _D), jnp.float32),
        [pltpu.VMEM((3, _C), jnp.int32) for _ in range(_NPK)],
        pltpu.VMEM_SHARED((_NPAD, _D), jnp.float32),
        [pltpu.SemaphoreType.DMA for _ in range(_NBUF)],
        pltpu.SemaphoreType.DMA,
        [pltpu.SemaphoreType.DMA for _ in range(_NPK)],
    ],
)
def _sc_gather_scatter(z_hbm, pk_hbm, g_out, gbufs, sbuf, pks, g_sh,
                       gsems, ssem, isems):
    cid = lax.axis_index("c")
    sid = lax.axis_index("s")
    wid = sid * _NC + cid
    base = sid * _RPT
    cbase = wid * _CHUNKS

    pltpu.async_copy(pk_hbm.at[cbase], pks[0], isems[0])
    pltpu.async_copy(pk_hbm.at[cbase + 1], pks[1], isems[1])

    zeros16 = jnp.zeros((16,), jnp.float32)

    def zero_g_body(i, carry):
        for q in range(_D // 16):
            sbuf[i, pl.ds(16 * q, 16)] = zeros16
        return carry

    lax.fori_loop(0, _C, zero_g_body, 0)
    for k in range(_RPT // _C):
        pltpu.sync_copy(sbuf, g_sh.at[pl.ds(base + _C * k, _C)])
    plsc.subcore_barrier()

    pltpu.make_async_copy(pk_hbm.at[cbase], pks[0], isems[0]).wait()
    for h in range(_GSPLIT):
        sl = pl.ds(h * (_C // _GSPLIT), _C // _GSPLIT)
        pltpu.async_copy(z_hbm.at[pl.ds(0, _C // _GSPLIT)], gbufs[0].at[sl], gsems[0])

    def quad_body(t, carry):
        for p in range(_NPK):
            j = t * _NPK + p
            pb = p % _NBUF
            pno = 1 - pb
            k0 = p
            k1 = (p + 1) % _NPK
            k2 = (p + 2) % _NPK
            k3 = (p + 3) % _NPK

            @pl.when(j + 2 < _CHUNKS)
            def _load_idx():
                pltpu.async_copy(pk_hbm.at[cbase + j + 2], pks[k2], isems[k2])

            @pl.when(j + 1 < _CHUNKS)
            def _issue_gather():
                pltpu.make_async_copy(
                    pk_hbm.at[cbase + j + 1], pks[k1], isems[k1]).wait()
                for h in range(_GSPLIT):
                    sl = pl.ds(h * (_C // _GSPLIT), _C // _GSPLIT)
                    pltpu.async_copy(
                        z_hbm.at[pl.ds(0, _C // _GSPLIT)], gbufs[pno].at[sl],
                        gsems[pno])

            for h in range(_GSPLIT):
                sl = pl.ds(h * (_C // _GSPLIT), _C // _GSPLIT)
                pltpu.make_async_copy(
                    z_hbm.at[pl.ds(0, _C // _GSPLIT)], gbufs[pb].at[sl],
                    gsems[pb]).wait()

            @pl.when(j >= 1)
            def _wait_prev_scatter():
                pltpu.make_async_copy(
                    sbuf, g_sh.at[pks[k3].at[0]], ssem).wait()

            gbuf = gbufs[pb]
            pk = pks[k0]

            def group_body(t2, c2):
                w16 = plsc.bitcast(pk[2, pl.ds(t2 * 16, 16)], jnp.float32)
                for l in range(16):
                    e = t2 * 16 + l
                    wv = jnp.full((16,), w16[l], jnp.float32)
                    for q in range(_D // 32):
                        x32 = gbuf[e, pl.ds(32 * q, 32)]
                        a, b = plsc.unpack(x32, format=plsc.PackFormat.INTERLEAVED)
                        sbuf[e, pl.ds(32 * q, 16)] = a * wv
                        sbuf[e, pl.ds(32 * q + 16, 16)] = b * wv
                return c2

            lax.fori_loop(0, _C // 16, group_body, 0)
            pltpu.async_copy(sbuf, g_sh.at[pk.at[0]], ssem, add=True)
        return carry

    lax.fori_loop(0, _CHUNKS // _NPK, quad_body, 0)
    lastk = (_CHUNKS - 1) % _NPK
    pltpu.make_async_copy(sbuf, g_sh.at[pks[lastk].at[0]], ssem).wait()
    plsc.subcore_barrier()
    pltpu.sync_copy(g_sh.at[pl.ds(base, _RPT)], g_out.at[cid].at[pl.ds(base, _RPT)])


_CD = 512
_DCH = _EPAD // (_NC * _NS * _CD)


@functools.partial(
    pl.kernel,
    out_type=jax.ShapeDtypeStruct((_NC, _NPAD, 16), jnp.float32),
    mesh=_mesh,
    compiler_params=pltpu.CompilerParams(
        use_tc_tiling_on_sc=False, needs_layout_passes=False),
    scratch_types=[
        [pltpu.VMEM((_CD // _C, _C), jnp.int32) for _ in range(2)],
        [pltpu.VMEM((_CD // _C, _C), jnp.float32) for _ in range(2)],
        [pltpu.VMEM((_CD, 16), jnp.float32) for _ in range(2)],
        pltpu.VMEM((_RPT, 16), jnp.float32),
        pltpu.VMEM_SHARED((_NPAD, 16), jnp.float32),
        [pltpu.SemaphoreType.DMA for _ in range(2)],
        [pltpu.SemaphoreType.DMA for _ in range(2)],
    ],
)
def _sc_degree(rows_hbm, w_hbm, d_out, rowbufs, wbufs, bufds, zd, d_sh,
               lsems, ssems):
    cid = lax.axis_index("c")
    sid = lax.axis_index("s")
    wid = sid * _NC + cid
    base = sid * _RPT
    nsub = _CD // _C
    cbase = wid * _DCH * nsub

    def _load(jj, p):
        pltpu.async_copy(
            rows_hbm.at[pl.ds(cbase + nsub * jj, nsub)], rowbufs[p], lsems[p])
        pltpu.async_copy(
            w_hbm.at[pl.ds(cbase + nsub * jj, nsub)], wbufs[p], lsems[p])

    def _wait_load(jj, p):
        pltpu.make_async_copy(
            rows_hbm.at[pl.ds(cbase + nsub * jj, nsub)], rowbufs[p],
            lsems[p]).wait()
        pltpu.make_async_copy(
            w_hbm.at[pl.ds(cbase + nsub * jj, nsub)], wbufs[p],
            lsems[p]).wait()

    def _wait_scatters(p):
        for h in range(nsub):
            pltpu.make_async_copy(
                bufds[p].at[pl.ds(_C * h, _C)],
                d_sh.at[rowbufs[p].at[h]], ssems[p]).wait()

    _load(0, 0)

    zeros16 = jnp.zeros((16,), jnp.float32)

    def zero_d_body(i, carry):
        zd[i, :] = zeros16
        return carry

    lax.fori_loop(0, _RPT, zero_d_body, 0)
    pltpu.sync_copy(zd, d_sh.at[pl.ds(base, _RPT)])
    plsc.subcore_barrier()

    def pair_body(tt, carry):
        for p in range(2):
            jj = tt * 2 + p
            pno = 1 - p

            @pl.when(jnp.logical_and(jj >= 1, jj + 1 < _DCH))
            def _drain_other():
                _wait_scatters(pno)

            @pl.when(jj + 1 < _DCH)
            def _load_next():
                _load(jj + 1, pno)

            _wait_load(jj, p)
            bufd = bufds[p]

            def group_body(t2, c2):
                h = t2 // (_C // 16)
                s = t2 % (_C // 16)
                w16 = wbufs[p][h, pl.ds(s * 16, 16)]
                for l in range(16):
                    bufd[t2 * 16 + l, :] = jnp.full((16,), w16[l], jnp.float32)
                return c2

            lax.fori_loop(0, _CD // 16, group_body, 0)
            for h in range(nsub):
                pltpu.async_copy(
                    bufd.at[pl.ds(_C * h, _C)],
                    d_sh.at[rowbufs[p].at[h]], ssems[p], add=True)
        return carry

    lax.fori_loop(0, _DCH // 2, pair_body, 0)
    _wait_scatters(0)
    _wait_scatters(1)
    plsc.subcore_barrier()
    pltpu.sync_copy(d_sh.at[pl.ds(base, _RPT)], d_out.at[cid].at[pl.ds(base, _RPT)])


_BLK = 1000


def _combine_body(x_ref, z_ref, g_ref, d_ref, o_ref):
    deg = d_ref[0, :, 0:1] + d_ref[1, :, 0:1]
    gsum = g_ref[0] + g_ref[1]
    pos = deg > 0.0
    inv = jnp.where(pos, 0.5 / jnp.where(pos, deg, 1.0), 0.0)
    h = jnp.where(pos, 0.5, 0.0)
    o_ref[...] = x_ref[...] - h * z_ref[...] + inv * gsum


_combine = pl.pallas_call(
    _combine_body,
    grid=(_N // _BLK,),
    in_specs=[
        pl.BlockSpec((_BLK, _D), lambda i: (i, 0)),
        pl.BlockSpec((_BLK, _D), lambda i: (i, 0)),
        pl.BlockSpec((_NC, _BLK, _D), lambda i: (0, i, 0)),
        pl.BlockSpec((_NC, _BLK, 16), lambda i: (0, i, 0)),
    ],
    out_specs=pl.BlockSpec((_BLK, _D), lambda i: (i, 0)),
    out_shape=jax.ShapeDtypeStruct((_N, _D), jnp.float32),
)


def kernel(x, z, edge_index, edge_weight):
    row = edge_index[0]
    col = edge_index[1]
    pad = _EPAD - _E
    rows2d = jnp.concatenate([row, jnp.zeros((pad,), jnp.int32)]).reshape(-1, _C)
    cols2d = jnp.concatenate([col, jnp.zeros((pad,), jnp.int32)]).reshape(-1, _C)
    w2d = jnp.concatenate([edge_weight, jnp.zeros((pad,), jnp.float32)]).reshape(-1, _C)
    wbits = lax.bitcast_convert_type(w2d, jnp.int32)
    pk = jnp.stack([rows2d, cols2d, wbits], axis=1)
    z_bf = z[:, _PERM].astype(jnp.bfloat16)
    g = _sc_gather_scatter(z_bf, pk)
    d = _sc_degree(rows2d, w2d)
    return _combine(x, z, g, d)


# speedup vs baseline: 1.5420x; 1.5420x over previous
import functools

import jax
import jax.numpy as jnp
from jax import lax
from jax.experimental import pallas as pl
from jax.experimental.pallas import tpu as pltpu
from jax.experimental.pallas import tpu_sc as plsc

_N = 10000
_D = 128
_E = 320000
_NC = 2
_NS = 16
_C = 128
_CHUNKS = 80
_NBUF = 2
_NPK = 4
_GSPLIT = 4
_EPAD = _NC * _NS * _CHUNKS * _C
_NPAD = 10240
_RPT = _NPAD // _NS

_mesh = plsc.VectorSubcoreMesh(core_axis_name="c", subcore_axis_name="s")


@functools.partial(
    pl.kernel,
    out_type=jax.ShapeDtypeStruct((_NC, _NPAD, _D), jnp.bfloat16),
    mesh=_mesh,
    compiler_params=pltpu.CompilerParams(
        use_tc_tiling_on_sc=False, needs_layout_passes=False),
    scratch_types=[
        [pltpu.VMEM((_C, _D), jnp.bfloat16) for _ in range(_NBUF)],
        pltpu.VMEM((_C, _D), jnp.bfloat16),
        [pltpu.VMEM((3, _C), jnp.int32) for _ in range(_NPK)],
        pltpu.VMEM_SHARED((_NPAD, _D), jnp.bfloat16),
        [pltpu.SemaphoreType.DMA for _ in range(_NBUF)],
        pltpu.SemaphoreType.DMA,
        [pltpu.SemaphoreType.DMA for _ in range(_NPK)],
    ],
)
def _sc_gather_scatter(z_hbm, pk_hbm, g_out, gbufs, sbuf, pks, g_sh,
                       gsems, ssem, isems):
    cid = lax.axis_index("c")
    sid = lax.axis_index("s")
    wid = sid * _NC + cid
    base = sid * _RPT
    cbase = wid * _CHUNKS

    pltpu.async_copy(pk_hbm.at[cbase], pks[0], isems[0])
    pltpu.async_copy(pk_hbm.at[cbase + 1], pks[1], isems[1])

    zeros32 = jnp.zeros((32,), jnp.bfloat16)

    def zero_g_body(i, carry):
        for q in range(_D // 32):
            sbuf[i, pl.ds(32 * q, 32)] = zeros32
        return carry

    lax.fori_loop(0, _C, zero_g_body, 0)
    for k in range(_RPT // _C):
        pltpu.sync_copy(sbuf, g_sh.at[pl.ds(base + _C * k, _C)])
    plsc.subcore_barrier()

    pltpu.make_async_copy(pk_hbm.at[cbase], pks[0], isems[0]).wait()
    for h in range(_GSPLIT):
        sl = pl.ds(h * (_C // _GSPLIT), _C // _GSPLIT)
        pltpu.async_copy(z_hbm.at[pks[0].at[1, sl]], gbufs[0].at[sl], gsems[0])

    def quad_body(t, carry):
        for p in range(_NPK):
            j = t * _NPK + p
            pb = p % _NBUF
            pno = 1 - pb
            k0 = p
            k1 = (p + 1) % _NPK
            k2 = (p + 2) % _NPK
            k3 = (p + 3) % _NPK

            @pl.when(j + 2 < _CHUNKS)
            def _load_idx():
                pltpu.async_copy(pk_hbm.at[cbase + j + 2], pks[k2], isems[k2])

            @pl.when(j + 1 < _CHUNKS)
            def _issue_gather():
                pltpu.make_async_copy(
                    pk_hbm.at[cbase + j + 1], pks[k1], isems[k1]).wait()
                for h in range(_GSPLIT):
                    sl = pl.ds(h * (_C // _GSPLIT), _C // _GSPLIT)
                    pltpu.async_copy(
                        z_hbm.at[pks[k1].at[1, sl]], gbufs[pno].at[sl],
                        gsems[pno])

            for h in range(_GSPLIT):
                sl = pl.ds(h * (_C // _GSPLIT), _C // _GSPLIT)
                pltpu.make_async_copy(
                    z_hbm.at[pks[k0].at[1, sl]], gbufs[pb].at[sl],
                    gsems[pb]).wait()

            @pl.when(j >= 1)
            def _wait_prev_scatter():
                pltpu.make_async_copy(
                    sbuf, g_sh.at[pks[k3].at[0]], ssem).wait()

            gbuf = gbufs[pb]
            pk = pks[k0]

            def group_body(t2, c2):
                w16 = plsc.bitcast(pk[2, pl.ds(t2 * 16, 16)], jnp.float32)
                for l in range(16):
                    e = t2 * 16 + l
                    wv = jnp.full((16,), w16[l], jnp.float32)
                    for q in range(_D // 32):
                        x32 = gbuf[e, pl.ds(32 * q, 32)]
                        a, b = plsc.unpack(x32, format=plsc.PackFormat.INTERLEAVED)
                        sbuf[e, pl.ds(32 * q, 32)] = plsc.pack(
                            a * wv, b * wv, format=plsc.PackFormat.INTERLEAVED)
                return c2

            lax.fori_loop(0, _C // 16, group_body, 0)
            pltpu.async_copy(sbuf, g_sh.at[pk.at[0]], ssem, add=True)
        return carry

    lax.fori_loop(0, _CHUNKS // _NPK, quad_body, 0)
    lastk = (_CHUNKS - 1) % _NPK
    pltpu.make_async_copy(sbuf, g_sh.at[pks[lastk].at[0]], ssem).wait()
    plsc.subcore_barrier()
    pltpu.sync_copy(g_sh.at[pl.ds(base, _RPT)], g_out.at[cid].at[pl.ds(base, _RPT)])


_CD = 512
_DCH = _EPAD // (_NC * _NS * _CD)


@functools.partial(
    pl.kernel,
    out_type=jax.ShapeDtypeStruct((_NC, _NPAD, 16), jnp.float32),
    mesh=_mesh,
    compiler_params=pltpu.CompilerParams(
        use_tc_tiling_on_sc=False, needs_layout_passes=False),
    scratch_types=[
        [pltpu.VMEM((_CD // _C, _C), jnp.int32) for _ in range(2)],
        [pltpu.VMEM((_CD // _C, _C), jnp.float32) for _ in range(2)],
        [pltpu.VMEM((_CD, 16), jnp.float32) for _ in range(2)],
        pltpu.VMEM((_RPT, 16), jnp.float32),
        pltpu.VMEM_SHARED((_NPAD, 16), jnp.float32),
        [pltpu.SemaphoreType.DMA for _ in range(2)],
        [pltpu.SemaphoreType.DMA for _ in range(2)],
    ],
)
def _sc_degree(rows_hbm, w_hbm, d_out, rowbufs, wbufs, bufds, zd, d_sh,
               lsems, ssems):
    cid = lax.axis_index("c")
    sid = lax.axis_index("s")
    wid = sid * _NC + cid
    base = sid * _RPT
    nsub = _CD // _C
    cbase = wid * _DCH * nsub

    def _load(jj, p):
        pltpu.async_copy(
            rows_hbm.at[pl.ds(cbase + nsub * jj, nsub)], rowbufs[p], lsems[p])
        pltpu.async_copy(
            w_hbm.at[pl.ds(cbase + nsub * jj, nsub)], wbufs[p], lsems[p])

    def _wait_load(jj, p):
        pltpu.make_async_copy(
            rows_hbm.at[pl.ds(cbase + nsub * jj, nsub)], rowbufs[p],
            lsems[p]).wait()
        pltpu.make_async_copy(
            w_hbm.at[pl.ds(cbase + nsub * jj, nsub)], wbufs[p],
            lsems[p]).wait()

    def _wait_scatters(p):
        for h in range(nsub):
            pltpu.make_async_copy(
                bufds[p].at[pl.ds(_C * h, _C)],
                d_sh.at[rowbufs[p].at[h]], ssems[p]).wait()

    _load(0, 0)

    zeros16 = jnp.zeros((16,), jnp.float32)

    def zero_d_body(i, carry):
        zd[i, :] = zeros16
        return carry

    lax.fori_loop(0, _RPT, zero_d_body, 0)
    pltpu.sync_copy(zd, d_sh.at[pl.ds(base, _RPT)])
    plsc.subcore_barrier()

    def pair_body(tt, carry):
        for p in range(2):
            jj = tt * 2 + p
            pno = 1 - p

            @pl.when(jnp.logical_and(jj >= 1, jj + 1 < _DCH))
            def _drain_other():
                _wait_scatters(pno)

            @pl.when(jj + 1 < _DCH)
            def _load_next():
                _load(jj + 1, pno)

            _wait_load(jj, p)
            bufd = bufds[p]

            def group_body(t2, c2):
                h = t2 // (_C // 16)
                s = t2 % (_C // 16)
                w16 = wbufs[p][h, pl.ds(s * 16, 16)]
                for l in range(16):
                    bufd[t2 * 16 + l, :] = jnp.full((16,), w16[l], jnp.float32)
                return c2

            lax.fori_loop(0, _CD // 16, group_body, 0)
            for h in range(nsub):
                pltpu.async_copy(
                    bufd.at[pl.ds(_C * h, _C)],
                    d_sh.at[rowbufs[p].at[h]], ssems[p], add=True)
        return carry

    lax.fori_loop(0, _DCH // 2, pair_body, 0)
    _wait_scatters(0)
    _wait_scatters(1)
    plsc.subcore_barrier()
    pltpu.sync_copy(d_sh.at[pl.ds(base, _RPT)], d_out.at[cid].at[pl.ds(base, _RPT)])


_BLK = 1000


def _combine_body(x_ref, z_ref, g_ref, d_ref, o_ref):
    deg = d_ref[0, :, 0:1] + d_ref[1, :, 0:1]
    gsum = g_ref[0].astype(jnp.float32) + g_ref[1].astype(jnp.float32)
    pos = deg > 0.0
    inv = jnp.where(pos, 0.5 / jnp.where(pos, deg, 1.0), 0.0)
    h = jnp.where(pos, 0.5, 0.0)
    o_ref[...] = x_ref[...] - h * z_ref[...] + inv * gsum


_combine = pl.pallas_call(
    _combine_body,
    grid=(_N // _BLK,),
    in_specs=[
        pl.BlockSpec((_BLK, _D), lambda i: (i, 0)),
        pl.BlockSpec((_BLK, _D), lambda i: (i, 0)),
        pl.BlockSpec((_NC, _BLK, _D), lambda i: (0, i, 0)),
        pl.BlockSpec((_NC, _BLK, 16), lambda i: (0, i, 0)),
    ],
    out_specs=pl.BlockSpec((_BLK, _D), lambda i: (i, 0)),
    out_shape=jax.ShapeDtypeStruct((_N, _D), jnp.float32),
)


def kernel(x, z, edge_index, edge_weight):
    row = edge_index[0]
    col = edge_index[1]
    pad = _EPAD - _E
    rows2d = jnp.concatenate([row, jnp.zeros((pad,), jnp.int32)]).reshape(-1, _C)
    cols2d = jnp.concatenate([col, jnp.zeros((pad,), jnp.int32)]).reshape(-1, _C)
    w2d = jnp.concatenate([edge_weight, jnp.zeros((pad,), jnp.float32)]).reshape(-1, _C)
    wbits = lax.bitcast_convert_type(w2d, jnp.int32)
    pk = jnp.stack([rows2d, cols2d, wbits], axis=1)
    z_bf = z.astype(jnp.bfloat16)
    g = _sc_gather_scatter(z_bf, pk)
    d = _sc_degree(rows2d, w2d)
    return _combine(x, z, g, d)
